# baseline (speedup 1.0000x reference)
import functools

import jax
import jax.numpy as jnp
from jax import lax
from jax.experimental import pallas as pl
from jax.experimental.pallas import tpu as pltpu
from jax.experimental.pallas import tpu_sc as plsc

N = 10000
D = 128
E = 320000
NC = 2
NS = 16
NW = NC * NS
N_PAD = 10240
RPT = N_PAD // NS
K = 128
SCH = 16
CH0 = 48
CH1 = 112
E_PAD = NS * K * (CH0 + CH1)
SINK = N
_F32 = jnp.float32


def _deg_body(dstp0_hbm, dstp1_hbm, out_hbm,
              dst_v, tab_v, red_v, out_v, shr_sh, sem):
    c = lax.axis_index("c")
    s = lax.axis_index("s")
    zeros = jnp.zeros((16,), _F32)

    def zero(i, carry):
        tab_v[pl.ds(i * 16, 16)] = zeros
        return carry

    lax.fori_loop(0, N_PAD // 16, zero, 0)

    def hist(j, carry):
        for l in range(K // 16):
            idx = dst_v[j, pl.ds(l * 16, 16)]
            cnt, last = plsc.scan_count(idx)
            plsc.addupdate_scatter(tab_v, [idx], cnt.astype(_F32), mask=last)
        return carry

    @pl.when(c == 0)
    def _():
        pltpu.sync_copy(dstp0_hbm.at[s], dst_v.at[pl.ds(0, CH0)])
        lax.fori_loop(0, CH0, hist, 0)

    @pl.when(c == 1)
    def _():
        pltpu.sync_copy(dstp1_hbm.at[s], dst_v)
        lax.fori_loop(0, CH1, hist, 0)

    pltpu.sync_copy(tab_v, shr_sh.at[s])
    plsc.subcore_barrier()
    pltpu.sync_copy(shr_sh.at[:, pl.ds(s * RPT, RPT)], red_v)

    def red(i, carry):
        v = red_v[0, pl.ds(i * 16, 16)]
        for t in range(1, NS):
            v = v + red_v[t, pl.ds(i * 16, 16)]
        out_v[pl.ds(i * 16, 16)] = v
        return carry

    lax.fori_loop(0, RPT // 16, red, 0)
    pltpu.sync_copy(out_v, out_hbm.at[c, pl.ds(s * RPT, RPT)])


NBUF = 2


def _prop_body(gs_hbm, srcp0_hbm, dstp0_hbm, srcp1_hbm, dstp1_hbm,
               zeros_hbm, out_hbm,
               src_v, dst_v, r0, r1, g0, g1, t0, t1, acc_sh):
    rows = (r0, r1)
    gsem = (g0, g1)
    tsem = (t0, t1)
    c = lax.axis_index("c")
    s = lax.axis_index("s")
    pltpu.sync_copy(zeros_hbm, acc_sh.at[pl.ds(s * RPT, RPT)])
    plsc.subcore_barrier()

    def run(srcp_hbm, dstp_hbm, nch):
        for h in range(nch // SCH):
            pltpu.sync_copy(srcp_hbm.at[s, pl.ds(h * SCH, SCH)], src_v)
            pltpu.sync_copy(dstp_hbm.at[s, pl.ds(h * SCH, SCH)], dst_v)
            for b in range(NBUF):
                pltpu.async_copy(gs_hbm.at[src_v.at[b]], rows[b], gsem[b])

            def chunk(i, carry):
                for b in range(NBUF):
                    j = i * NBUF + b
                    pltpu.make_async_copy(
                        gs_hbm.at[src_v.at[j]], rows[b], gsem[b]).wait()
                    pltpu.async_copy(
                        rows[b], acc_sh.at[dst_v.at[j]], tsem[b], add=True)
                for b in range(NBUF):
                    j = i * NBUF + b
                    pltpu.make_async_copy(
                        rows[b], acc_sh.at[dst_v.at[j]], tsem[b]).wait()
                    pltpu.async_copy(
                        gs_hbm.at[src_v.at[j + NBUF]], rows[b], gsem[b])
                return carry

            lax.fori_loop(0, SCH // NBUF - 1, chunk, 0)
            for b in range(NBUF):
                j = SCH - NBUF + b
                pltpu.make_async_copy(
                    gs_hbm.at[src_v.at[j]], rows[b], gsem[b]).wait()
                pltpu.async_copy(
                    rows[b], acc_sh.at[dst_v.at[j]], tsem[b], add=True)
            for b in range(NBUF):
                j = SCH - NBUF + b
                pltpu.make_async_copy(
                    rows[b], acc_sh.at[dst_v.at[j]], tsem[b]).wait()

    @pl.when(c == 0)
    def _():
        run(srcp0_hbm, dstp0_hbm, CH0)

    @pl.when(c == 1)
    def _():
        run(srcp1_hbm, dstp1_hbm, CH1)

    plsc.subcore_barrier()
    pltpu.sync_copy(acc_sh.at[pl.ds(s * RPT, RPT)],
                    out_hbm.at[c, pl.ds(s * RPT, RPT)])


def _sc_calls():
    mesh = plsc.VectorSubcoreMesh(core_axis_name="c", subcore_axis_name="s")
    deg_call = pl.kernel(
        _deg_body,
        out_type=jax.ShapeDtypeStruct((NC, N_PAD), _F32),
        mesh=mesh,
        compiler_params=pltpu.CompilerParams(needs_layout_passes=False),
        scratch_types=[
            pltpu.VMEM((CH1, K), jnp.int32),
            pltpu.VMEM((N_PAD,), _F32),
            pltpu.VMEM((NS, RPT), _F32),
            pltpu.VMEM((RPT,), _F32),
            pltpu.VMEM_SHARED((NS, N_PAD), _F32),
            pltpu.SemaphoreType.DMA,
        ],
    )
    prop_call = pl.kernel(
        _prop_body,
        out_type=jax.ShapeDtypeStruct((NC, N_PAD, D), _F32),
        mesh=mesh,
        scratch_types=(
            [pltpu.VMEM((SCH, K), jnp.int32)] * 2
            + [pltpu.VMEM((K, D), _F32)] * NBUF
            + [pltpu.SemaphoreType.DMA] * (2 * NBUF)
            + [pltpu.VMEM_SHARED((N_PAD, D), _F32)]
        ),
    )
    return deg_call, prop_call


def _row_mask():
    return (lax.broadcasted_iota(jnp.int32, (N_PAD, 1), 0) < N).astype(_F32)


def _dinv(deg):
    return jnp.where(deg > 0, lax.rsqrt(deg), 0.0)


def _bn(h, mask, g, b):
    m = jnp.sum(h, axis=0) / N
    v = jnp.sum(mask * (h - m) ** 2, axis=0) / N
    return (h - m) / jnp.sqrt(v + 1e-5) * g + b


def _dense1_body(x_ref, deg_ref, bn1g, bn1b, l1w, l1b, bn2g, bn2b, l2w,
                 gs2_ref):
    mask = _row_mask()
    x = x_ref[...] * mask
    h = _bn(x, mask, bn1g[...], bn1b[...])
    h = jnp.maximum(
        jnp.dot(h, l1w[...], preferred_element_type=_F32) + l1b[...], 0.0)
    h = h * mask
    h = _bn(h, mask, bn2g[...], bn2b[...])
    g2 = jnp.dot(h, l2w[...], preferred_element_type=_F32)
    gs2_ref[...] = g2 * _dinv(deg_ref[...]) * mask


def _dense2_body(acc_ref, gs2_ref, deg_ref, l2b, bn3g, bn3b, l3w, gs3_ref):
    mask = _row_mask()
    dinv = _dinv(deg_ref[...])
    a = acc_ref[0] + acc_ref[1] + gs2_ref[...]
    h = jnp.maximum(a * dinv + l2b[...], 0.0) * mask
    h = _bn(h, mask, bn3g[...], bn3b[...])
    g3 = jnp.dot(h, l3w[...], preferred_element_type=_F32)
    gs3_ref[...] = g3 * dinv * mask


def _final_body(acc_ref, gs3_ref, deg_ref, l3b, out_ref):
    dinv = _dinv(deg_ref[...])
    out_ref[...] = (acc_ref[0] + acc_ref[1] + gs3_ref[...]) * dinv + l3b[...]


_dense1 = pl.pallas_call(
    _dense1_body, out_shape=jax.ShapeDtypeStruct((N_PAD, D), _F32))
_dense2 = pl.pallas_call(
    _dense2_body, out_shape=jax.ShapeDtypeStruct((N_PAD, D), _F32))
_final = pl.pallas_call(
    _final_body, out_shape=jax.ShapeDtypeStruct((N_PAD, D), _F32))


def kernel(x, edge_index, bn1_g, bn1_b, lin1_W, lin1_b,
           bn2_g, bn2_b, lin2_W, lin2_b, bn3_g, bn3_b, lin3_W, lin3_b):
    deg_call, prop_call = _sc_calls()
    src = edge_index[0]
    dst = edge_index[1]
    pad = jnp.full((E_PAD - E,), SINK, jnp.int32)
    srcf = jnp.concatenate([src, pad])
    dstf = jnp.concatenate([dst, pad])
    n0 = NS * CH0 * K
    srcp0 = srcf[:n0].reshape(NS, CH0, K)
    dstp0 = dstf[:n0].reshape(NS, CH0, K)
    srcp1 = srcf[n0:].reshape(NS, CH1, K)
    dstp1 = dstf[n0:].reshape(NS, CH1, K)
    zrows = jnp.zeros((RPT, D), _F32)
    xp = jnp.concatenate([x, jnp.zeros((N_PAD - N, D), _F32)])

    degparts = deg_call(dstp0, dstp1)
    deg = (degparts[0] + degparts[1] + 1.0).reshape(N_PAD, 1)
    gs2 = _dense1(xp, deg, bn1_g, bn1_b, lin1_W, lin1_b, bn2_g, bn2_b, lin2_W)
    acc2 = prop_call(gs2, srcp0, dstp0, srcp1, dstp1, zrows)
    gs3 = _dense2(acc2, gs2, deg, lin2_b, bn3_g, bn3_b, lin3_W)
    acc3 = prop_call(gs3, srcp0, dstp0, srcp1, dstp1, zrows)
    out = _final(acc3, gs3, deg, lin3_b)
    return out[:N]


# speedup vs baseline: 1.1439x; 1.1439x over previous
import functools

import jax
import jax.numpy as jnp
from jax import lax
from jax.experimental import pallas as pl
from jax.experimental.pallas import tpu as pltpu
from jax.experimental.pallas import tpu_sc as plsc

N = 10000
D = 128
E = 320000
NC = 2
NS = 16
NW = NC * NS
N_PAD = 10240
RPT = N_PAD // NS
K = 128
SCH = 16
CH0 = 112
CH1 = 48
E_PAD = NS * K * (CH0 + CH1)
SINK = N
_F32 = jnp.float32


def _deg_body(dstp0_hbm, dstp1_hbm, out_hbm,
              dst_v, tab_v, red_v, out_v, shr_sh, sem):
    c = lax.axis_index("c")
    s = lax.axis_index("s")
    zeros = jnp.zeros((16,), _F32)

    def zero(i, carry):
        tab_v[pl.ds(i * 16, 16)] = zeros
        return carry

    lax.fori_loop(0, N_PAD // 16, zero, 0)

    def hist(j, carry):
        for l in range(K // 16):
            idx = dst_v[j, pl.ds(l * 16, 16)]
            cnt, last = plsc.scan_count(idx)
            plsc.addupdate_scatter(tab_v, [idx], cnt.astype(_F32), mask=last)
        return carry

    @pl.when(c == 0)
    def _():
        pltpu.sync_copy(dstp0_hbm.at[s], dst_v.at[pl.ds(0, CH0)])
        lax.fori_loop(0, CH0, hist, 0)

    @pl.when(c == 1)
    def _():
        pltpu.sync_copy(dstp1_hbm.at[s], dst_v.at[pl.ds(0, CH1)])
        lax.fori_loop(0, CH1, hist, 0)

    pltpu.sync_copy(tab_v, shr_sh.at[s])
    plsc.subcore_barrier()
    pltpu.sync_copy(shr_sh.at[:, pl.ds(s * RPT, RPT)], red_v)

    def red(i, carry):
        v = red_v[0, pl.ds(i * 16, 16)]
        for t in range(1, NS):
            v = v + red_v[t, pl.ds(i * 16, 16)]
        out_v[pl.ds(i * 16, 16)] = v
        return carry

    lax.fori_loop(0, RPT // 16, red, 0)
    pltpu.sync_copy(out_v, out_hbm.at[c, pl.ds(s * RPT, RPT)])


NBUF = 2


def _prop_body(gs_hbm, srcp0_hbm, dstp0_hbm, srcp1_hbm, dstp1_hbm,
               zeros_hbm, out_hbm,
               src_v, dst_v, r0, r1, g0, g1, t0, t1, acc_sh):
    rows = (r0, r1)
    gsem = (g0, g1)
    tsem = (t0, t1)
    c = lax.axis_index("c")
    s = lax.axis_index("s")
    pltpu.sync_copy(zeros_hbm, acc_sh.at[pl.ds(s * RPT, RPT)])
    plsc.subcore_barrier()

    def run(srcp_hbm, dstp_hbm, nch):
        for h in range(nch // SCH):
            pltpu.sync_copy(srcp_hbm.at[s, pl.ds(h * SCH, SCH)], src_v)
            pltpu.sync_copy(dstp_hbm.at[s, pl.ds(h * SCH, SCH)], dst_v)
            for b in range(NBUF):
                pltpu.async_copy(gs_hbm.at[src_v.at[b]], rows[b], gsem[b])

            def chunk(i, carry):
                for b in range(NBUF):
                    j = i * NBUF + b
                    pltpu.make_async_copy(
                        gs_hbm.at[src_v.at[j]], rows[b], gsem[b]).wait()
                    pltpu.async_copy(
                        rows[b], acc_sh.at[dst_v.at[j]], tsem[b], add=True)
                for b in range(NBUF):
                    j = i * NBUF + b
                    pltpu.make_async_copy(
                        rows[b], acc_sh.at[dst_v.at[j]], tsem[b]).wait()
                    pltpu.async_copy(
                        gs_hbm.at[src_v.at[j + NBUF]], rows[b], gsem[b])
                return carry

            lax.fori_loop(0, SCH // NBUF - 1, chunk, 0)
            for b in range(NBUF):
                j = SCH - NBUF + b
                pltpu.make_async_copy(
                    gs_hbm.at[src_v.at[j]], rows[b], gsem[b]).wait()
                pltpu.async_copy(
                    rows[b], acc_sh.at[dst_v.at[j]], tsem[b], add=True)
            for b in range(NBUF):
                j = SCH - NBUF + b
                pltpu.make_async_copy(
                    rows[b], acc_sh.at[dst_v.at[j]], tsem[b]).wait()

    @pl.when(c == 0)
    def _():
        run(srcp0_hbm, dstp0_hbm, CH0)

    @pl.when(c == 1)
    def _():
        run(srcp1_hbm, dstp1_hbm, CH1)

    plsc.subcore_barrier()
    pltpu.sync_copy(acc_sh.at[pl.ds(s * RPT, RPT)],
                    out_hbm.at[c, pl.ds(s * RPT, RPT)])


def _sc_calls():
    mesh = plsc.VectorSubcoreMesh(core_axis_name="c", subcore_axis_name="s")
    deg_call = pl.kernel(
        _deg_body,
        out_type=jax.ShapeDtypeStruct((NC, N_PAD), _F32),
        mesh=mesh,
        compiler_params=pltpu.CompilerParams(needs_layout_passes=False),
        scratch_types=[
            pltpu.VMEM((max(CH0, CH1), K), jnp.int32),
            pltpu.VMEM((N_PAD,), _F32),
            pltpu.VMEM((NS, RPT), _F32),
            pltpu.VMEM((RPT,), _F32),
            pltpu.VMEM_SHARED((NS, N_PAD), _F32),
            pltpu.SemaphoreType.DMA,
        ],
    )
    prop_call = pl.kernel(
        _prop_body,
        out_type=jax.ShapeDtypeStruct((NC, N_PAD, D), _F32),
        mesh=mesh,
        scratch_types=(
            [pltpu.VMEM((SCH, K), jnp.int32)] * 2
            + [pltpu.VMEM((K, D), _F32)] * NBUF
            + [pltpu.SemaphoreType.DMA] * (2 * NBUF)
            + [pltpu.VMEM_SHARED((N_PAD, D), _F32)]
        ),
    )
    return deg_call, prop_call


def _row_mask():
    return (lax.broadcasted_iota(jnp.int32, (N_PAD, 1), 0) < N).astype(_F32)


def _dinv(deg):
    return jnp.where(deg > 0, lax.rsqrt(deg), 0.0)


def _bn(h, mask, g, b):
    m = jnp.sum(h, axis=0) / N
    v = jnp.sum(mask * (h - m) ** 2, axis=0) / N
    return (h - m) / jnp.sqrt(v + 1e-5) * g + b


def _dense1_body(x_ref, deg_ref, bn1g, bn1b, l1w, l1b, bn2g, bn2b, l2w,
                 gs2_ref):
    mask = _row_mask()
    x = x_ref[...] * mask
    h = _bn(x, mask, bn1g[...], bn1b[...])
    h = jnp.maximum(
        jnp.dot(h, l1w[...], preferred_element_type=_F32) + l1b[...], 0.0)
    h = h * mask
    h = _bn(h, mask, bn2g[...], bn2b[...])
    g2 = jnp.dot(h, l2w[...], preferred_element_type=_F32)
    gs2_ref[...] = g2 * _dinv(deg_ref[...]) * mask


def _dense2_body(acc_ref, gs2_ref, deg_ref, l2b, bn3g, bn3b, l3w, gs3_ref):
    mask = _row_mask()
    dinv = _dinv(deg_ref[...])
    a = acc_ref[0] + acc_ref[1] + gs2_ref[...]
    h = jnp.maximum(a * dinv + l2b[...], 0.0) * mask
    h = _bn(h, mask, bn3g[...], bn3b[...])
    g3 = jnp.dot(h, l3w[...], preferred_element_type=_F32)
    gs3_ref[...] = g3 * dinv * mask


def _final_body(acc_ref, gs3_ref, deg_ref, l3b, out_ref):
    dinv = _dinv(deg_ref[...])
    out_ref[...] = (acc_ref[0] + acc_ref[1] + gs3_ref[...]) * dinv + l3b[...]


_dense1 = pl.pallas_call(
    _dense1_body, out_shape=jax.ShapeDtypeStruct((N_PAD, D), _F32))
_dense2 = pl.pallas_call(
    _dense2_body, out_shape=jax.ShapeDtypeStruct((N_PAD, D), _F32))
_final = pl.pallas_call(
    _final_body, out_shape=jax.ShapeDtypeStruct((N_PAD, D), _F32))


def kernel(x, edge_index, bn1_g, bn1_b, lin1_W, lin1_b,
           bn2_g, bn2_b, lin2_W, lin2_b, bn3_g, bn3_b, lin3_W, lin3_b):
    deg_call, prop_call = _sc_calls()
    src = edge_index[0]
    dst = edge_index[1]
    pad = jnp.full((E_PAD - E,), SINK, jnp.int32)
    srcf = jnp.concatenate([src, pad])
    dstf = jnp.concatenate([dst, pad])
    n0 = NS * CH0 * K
    srcp0 = srcf[:n0].reshape(NS, CH0, K)
    dstp0 = dstf[:n0].reshape(NS, CH0, K)
    srcp1 = srcf[n0:].reshape(NS, CH1, K)
    dstp1 = dstf[n0:].reshape(NS, CH1, K)
    zrows = jnp.zeros((RPT, D), _F32)
    xp = jnp.concatenate([x, jnp.zeros((N_PAD - N, D), _F32)])

    degparts = deg_call(dstp0, dstp1)
    deg = (degparts[0] + degparts[1] + 1.0).reshape(N_PAD, 1)
    gs2 = _dense1(xp, deg, bn1_g, bn1_b, lin1_W, lin1_b, bn2_g, bn2_b, lin2_W)
    acc2 = prop_call(gs2, srcp0, dstp0, srcp1, dstp1, zrows)
    gs3 = _dense2(acc2, gs2, deg, lin2_b, bn3_g, bn3_b, lin3_W)
    acc3 = prop_call(gs3, srcp0, dstp0, srcp1, dstp1, zrows)
    out = _final(acc3, gs3, deg, lin3_b)
    return out[:N]


# speedup vs baseline: 1.2938x; 1.1311x over previous
import functools

import jax
import jax.numpy as jnp
from jax import lax
from jax.experimental import pallas as pl
from jax.experimental.pallas import tpu as pltpu
from jax.experimental.pallas import tpu_sc as plsc

N = 10000
D = 128
E = 320000
NC = 2
NS = 16
NW = NC * NS
N_PAD = 10240
RPT = N_PAD // NS
K = 128
SCH = 8
CH0 = 152
CH1 = 8
E_PAD = NS * K * (CH0 + CH1)
SINK = N
_F32 = jnp.float32


def _deg_body(dstp0_hbm, dstp1_hbm, out_hbm,
              dst_v, tab_v, red_v, out_v, shr_sh, sem):
    c = lax.axis_index("c")
    s = lax.axis_index("s")
    zeros = jnp.zeros((16,), _F32)

    def zero(i, carry):
        tab_v[pl.ds(i * 16, 16)] = zeros
        return carry

    lax.fori_loop(0, N_PAD // 16, zero, 0)

    def hist(j, carry):
        for l in range(K // 16):
            idx = dst_v[j, pl.ds(l * 16, 16)]
            cnt, last = plsc.scan_count(idx)
            plsc.addupdate_scatter(tab_v, [idx], cnt.astype(_F32), mask=last)
        return carry

    @pl.when(c == 0)
    def _():
        pltpu.sync_copy(dstp0_hbm.at[s], dst_v.at[pl.ds(0, CH0)])
        lax.fori_loop(0, CH0, hist, 0)

    @pl.when(c == 1)
    def _():
        pltpu.sync_copy(dstp1_hbm.at[s], dst_v.at[pl.ds(0, CH1)])
        lax.fori_loop(0, CH1, hist, 0)

    pltpu.sync_copy(tab_v, shr_sh.at[s])
    plsc.subcore_barrier()
    pltpu.sync_copy(shr_sh.at[:, pl.ds(s * RPT, RPT)], red_v)

    def red(i, carry):
        v = red_v[0, pl.ds(i * 16, 16)]
        for t in range(1, NS):
            v = v + red_v[t, pl.ds(i * 16, 16)]
        out_v[pl.ds(i * 16, 16)] = v
        return carry

    lax.fori_loop(0, RPT // 16, red, 0)
    pltpu.sync_copy(out_v, out_hbm.at[c, pl.ds(s * RPT, RPT)])


NBUF = 2


def _prop_body(gs_hbm, srcp0_hbm, dstp0_hbm, srcp1_hbm, dstp1_hbm,
               zeros_hbm, out_hbm,
               src_v, dst_v, r0, r1, g0, g1, t0, t1, acc_sh):
    rows = (r0, r1)
    gsem = (g0, g1)
    tsem = (t0, t1)
    c = lax.axis_index("c")
    s = lax.axis_index("s")
    pltpu.sync_copy(zeros_hbm, acc_sh.at[pl.ds(s * RPT, RPT)])
    plsc.subcore_barrier()

    def run(srcp_hbm, dstp_hbm, nch):
        for h in range(nch // SCH):
            pltpu.sync_copy(srcp_hbm.at[s, pl.ds(h * SCH, SCH)], src_v)
            pltpu.sync_copy(dstp_hbm.at[s, pl.ds(h * SCH, SCH)], dst_v)
            for b in range(NBUF):
                pltpu.async_copy(gs_hbm.at[src_v.at[b]], rows[b], gsem[b])

            def chunk(i, carry):
                for b in range(NBUF):
                    j = i * NBUF + b
                    pltpu.make_async_copy(
                        gs_hbm.at[src_v.at[j]], rows[b], gsem[b]).wait()
                    pltpu.async_copy(
                        rows[b], acc_sh.at[dst_v.at[j]], tsem[b], add=True)
                for b in range(NBUF):
                    j = i * NBUF + b
                    pltpu.make_async_copy(
                        rows[b], acc_sh.at[dst_v.at[j]], tsem[b]).wait()
                    pltpu.async_copy(
                        gs_hbm.at[src_v.at[j + NBUF]], rows[b], gsem[b])
                return carry

            lax.fori_loop(0, SCH // NBUF - 1, chunk, 0)
            for b in range(NBUF):
                j = SCH - NBUF + b
                pltpu.make_async_copy(
                    gs_hbm.at[src_v.at[j]], rows[b], gsem[b]).wait()
                pltpu.async_copy(
                    rows[b], acc_sh.at[dst_v.at[j]], tsem[b], add=True)
            for b in range(NBUF):
                j = SCH - NBUF + b
                pltpu.make_async_copy(
                    rows[b], acc_sh.at[dst_v.at[j]], tsem[b]).wait()

    @pl.when(c == 0)
    def _():
        run(srcp0_hbm, dstp0_hbm, CH0)

    @pl.when(c == 1)
    def _():
        run(srcp1_hbm, dstp1_hbm, CH1)

    plsc.subcore_barrier()
    pltpu.sync_copy(acc_sh.at[pl.ds(s * RPT, RPT)],
                    out_hbm.at[c, pl.ds(s * RPT, RPT)])


def _sc_calls():
    mesh = plsc.VectorSubcoreMesh(core_axis_name="c", subcore_axis_name="s")
    deg_call = pl.kernel(
        _deg_body,
        out_type=jax.ShapeDtypeStruct((NC, N_PAD), _F32),
        mesh=mesh,
        compiler_params=pltpu.CompilerParams(needs_layout_passes=False),
        scratch_types=[
            pltpu.VMEM((max(CH0, CH1), K), jnp.int32),
            pltpu.VMEM((N_PAD,), _F32),
            pltpu.VMEM((NS, RPT), _F32),
            pltpu.VMEM((RPT,), _F32),
            pltpu.VMEM_SHARED((NS, N_PAD), _F32),
            pltpu.SemaphoreType.DMA,
        ],
    )
    prop_call = pl.kernel(
        _prop_body,
        out_type=jax.ShapeDtypeStruct((NC, N_PAD, D), _F32),
        mesh=mesh,
        scratch_types=(
            [pltpu.VMEM((SCH, K), jnp.int32)] * 2
            + [pltpu.VMEM((K, D), _F32)] * NBUF
            + [pltpu.SemaphoreType.DMA] * (2 * NBUF)
            + [pltpu.VMEM_SHARED((N_PAD, D), _F32)]
        ),
    )
    return deg_call, prop_call


def _row_mask():
    return (lax.broadcasted_iota(jnp.int32, (N_PAD, 1), 0) < N).astype(_F32)


def _dinv(deg):
    return jnp.where(deg > 0, lax.rsqrt(deg), 0.0)


def _bn(h, mask, g, b):
    m = jnp.sum(h, axis=0) / N
    v = jnp.sum(mask * (h - m) ** 2, axis=0) / N
    return (h - m) / jnp.sqrt(v + 1e-5) * g + b


def _dense1_body(x_ref, deg_ref, bn1g, bn1b, l1w, l1b, bn2g, bn2b, l2w,
                 gs2_ref):
    mask = _row_mask()
    x = x_ref[...] * mask
    h = _bn(x, mask, bn1g[...], bn1b[...])
    h = jnp.maximum(
        jnp.dot(h, l1w[...], preferred_element_type=_F32) + l1b[...], 0.0)
    h = h * mask
    h = _bn(h, mask, bn2g[...], bn2b[...])
    g2 = jnp.dot(h, l2w[...], preferred_element_type=_F32)
    gs2_ref[...] = g2 * _dinv(deg_ref[...]) * mask


def _dense2_body(acc_ref, gs2_ref, deg_ref, l2b, bn3g, bn3b, l3w, gs3_ref):
    mask = _row_mask()
    dinv = _dinv(deg_ref[...])
    a = acc_ref[0] + acc_ref[1] + gs2_ref[...]
    h = jnp.maximum(a * dinv + l2b[...], 0.0) * mask
    h = _bn(h, mask, bn3g[...], bn3b[...])
    g3 = jnp.dot(h, l3w[...], preferred_element_type=_F32)
    gs3_ref[...] = g3 * dinv * mask


def _final_body(acc_ref, gs3_ref, deg_ref, l3b, out_ref):
    dinv = _dinv(deg_ref[...])
    out_ref[...] = (acc_ref[0] + acc_ref[1] + gs3_ref[...]) * dinv + l3b[...]


_dense1 = pl.pallas_call(
    _dense1_body, out_shape=jax.ShapeDtypeStruct((N_PAD, D), _F32))
_dense2 = pl.pallas_call(
    _dense2_body, out_shape=jax.ShapeDtypeStruct((N_PAD, D), _F32))
_final = pl.pallas_call(
    _final_body, out_shape=jax.ShapeDtypeStruct((N_PAD, D), _F32))


def kernel(x, edge_index, bn1_g, bn1_b, lin1_W, lin1_b,
           bn2_g, bn2_b, lin2_W, lin2_b, bn3_g, bn3_b, lin3_W, lin3_b):
    deg_call, prop_call = _sc_calls()
    src = edge_index[0]
    dst = edge_index[1]
    pad = jnp.full((E_PAD - E,), SINK, jnp.int32)
    srcf = jnp.concatenate([src, pad])
    dstf = jnp.concatenate([dst, pad])
    n0 = NS * CH0 * K
    srcp0 = srcf[:n0].reshape(NS, CH0, K)
    dstp0 = dstf[:n0].reshape(NS, CH0, K)
    srcp1 = srcf[n0:].reshape(NS, CH1, K)
    dstp1 = dstf[n0:].reshape(NS, CH1, K)
    zrows = jnp.zeros((RPT, D), _F32)
    xp = jnp.concatenate([x, jnp.zeros((N_PAD - N, D), _F32)])

    degparts = deg_call(dstp0, dstp1)
    deg = (degparts[0] + degparts[1] + 1.0).reshape(N_PAD, 1)
    gs2 = _dense1(xp, deg, bn1_g, bn1_b, lin1_W, lin1_b, bn2_g, bn2_b, lin2_W)
    acc2 = prop_call(gs2, srcp0, dstp0, srcp1, dstp1, zrows)
    gs3 = _dense2(acc2, gs2, deg, lin2_b, bn3_g, bn3_b, lin3_W)
    acc3 = prop_call(gs3, srcp0, dstp0, srcp1, dstp1, zrows)
    out = _final(acc3, gs3, deg, lin3_b)
    return out[:N]
